# Initial kernel scaffold; baseline (speedup 1.0000x reference)
import jax
import jax.numpy as jnp
from jax.experimental import pallas as pl


def kernel(feats, segment_ids, W, b):
    raise NotImplementedError("write your pallas kernel here")


# speedup vs baseline: 3.1904x; 3.1904x over previous
import functools

import jax
import jax.numpy as jnp
from jax import lax
from jax.experimental import pallas as pl
from jax.experimental.pallas import tpu as pltpu
from jax.experimental.pallas import tpu_sc as plsc

L = 16
D = 128
DV = D // L
NSEG = 1024
NC = 2
NS = 16
NW = NC * NS
SEG_PER_W = NSEG // NW
CHUNK = 128


def _sc_kernel(feats, starts, wvec, bvec):
    n_rows = feats.shape[0]

    mesh = plsc.VectorSubcoreMesh(core_axis_name="c", subcore_axis_name="s")

    @functools.partial(
        pl.kernel,
        mesh=mesh,
        out_type=jax.ShapeDtypeStruct((NSEG, 2 * D), jnp.float32),
        scratch_types=[
            pltpu.VMEM((starts.shape[0],), jnp.int32),
            pltpu.VMEM((D,), jnp.float32),
            pltpu.VMEM((L,), jnp.float32),
            pltpu.VMEM((CHUNK, D), jnp.float32),
            pltpu.VMEM((SEG_PER_W, 2 * D), jnp.float32),
        ],
        compiler_params=pltpu.CompilerParams(needs_layout_passes=False),
    )
    def body(feats_hbm, starts_hbm, w_hbm, b_hbm, out_hbm,
             starts_v, w_v, b_v, buf, out_stage):
        wid = lax.axis_index("s") * NC + lax.axis_index("c")
        pltpu.sync_copy(starts_hbm, starts_v)
        pltpu.sync_copy(w_hbm, w_v)
        pltpu.sync_copy(b_hbm, b_v)

        wreg = [w_v[pl.ds(t * L, L)] for t in range(DV)]
        breg = b_v[...]

        seg0 = wid * SEG_PER_W

        def seg_body(sl, _):
            s = seg0 + sl
            sv = starts_v[pl.ds(s, L)]
            st = sv[0]
            en = sv[1]

            zero = jnp.zeros((L,), jnp.float32)
            ninf = jnp.full((L,), -jnp.inf, jnp.float32)
            init = tuple(zero for _ in range(DV)) + tuple(
                ninf for _ in range(DV))

            a0 = st & (-8)
            nch = (en - a0 + (CHUNK - 1)) // CHUNK

            def chunk_body(i, carry):
                accs = list(carry[:DV])
                accm = list(carry[DV:])
                base = a0 + i * CHUNK
                cs = pl.multiple_of(jnp.minimum(base, n_rows - CHUNK), 8)
                pltpu.sync_copy(feats_hbm.at[pl.ds(cs, CHUNK)], buf)
                lo = jnp.maximum(st, base)
                hi = jnp.minimum(en, base + CHUNK)
                off = lo - cs
                k = hi - lo

                def row_body(j, rc):
                    raccs = list(rc[:DV])
                    raccm = list(rc[DV:])
                    row = off + j
                    rv = [buf[row, pl.ds(t * L, L)] for t in range(DV)]
                    part = rv[0] * wreg[0]
                    for t in range(1, DV):
                        part = part + rv[t] * wreg[t]
                    dot = jnp.sum(part)
                    x = jnp.broadcast_to(dot, (L,)) + breg
                    gate = 1.0 / (1.0 + jnp.exp(-x))
                    for t in range(DV):
                        raccs[t] = raccs[t] + gate * rv[t]
                        raccm[t] = jnp.maximum(raccm[t], rv[t])
                    return tuple(raccs) + tuple(raccm)

                res = lax.fori_loop(0, k, row_body, tuple(accs) + tuple(accm))
                return tuple(res)

            fin = lax.fori_loop(0, nch, chunk_body, init)
            for t in range(DV):
                out_stage[sl, pl.ds(t * L, L)] = fin[t]
                out_stage[sl, pl.ds(D + t * L, L)] = fin[DV + t]
            return 0

        lax.fori_loop(0, SEG_PER_W, seg_body, 0)
        pltpu.sync_copy(out_stage, out_hbm.at[pl.ds(seg0, SEG_PER_W)])

    return body(feats, starts, wvec, bvec)


def kernel(feats, segment_ids, W, b):
    starts = jnp.searchsorted(
        segment_ids, jnp.arange(NSEG + 1, dtype=segment_ids.dtype)
    ).astype(jnp.int32)
    starts = jnp.pad(starts, (0, L - 1))
    wvec = W.reshape(D).astype(jnp.float32)
    bvec = jnp.broadcast_to(b.astype(jnp.float32), (L,))
    return _sc_kernel(feats, starts, wvec, bvec)
